# Initial kernel scaffold; baseline (speedup 1.0000x reference)
import jax
import jax.numpy as jnp
from jax.experimental import pallas as pl


def kernel(x):
    raise NotImplementedError("write your pallas kernel here")


# speedup vs baseline: 3.6552x; 3.6552x over previous
import jax
import jax.numpy as jnp
from jax.experimental import pallas as pl
from jax.experimental.pallas import tpu as pltpu

N_FFT = 2048
HOP = 512
OVERLAP = N_FFT // HOP
PREV = OVERLAP - 1
BF = 32


def _frame_kernel(a0_ref, a1_ref, a2_ref, main_ref, o_ref):
    k = pl.program_id(1)
    rows = jnp.concatenate(
        [a0_ref[0, 0], a1_ref[0, 0], a2_ref[0, 0], main_ref[0]], axis=0
    )
    rid = jax.lax.broadcasted_iota(jnp.int32, rows.shape, 0)
    rows = jnp.where((k == 0) & (rid < PREV), 0.0, rows)
    o_ref[0] = jnp.concatenate(
        [rows[s:s + BF] for s in range(OVERLAP)], axis=1
    )


def kernel(x):
    B, T = x.shape
    num_chunks = T // HOP
    num_frames = num_chunks
    x3 = x.reshape(B, num_chunks, HOP)
    x4 = x.reshape(B, num_chunks, 1, HOP)
    grid = (B, num_frames // BF)

    def prev_map(i):
        return lambda b, k: (b, jnp.maximum(BF * k - PREV + i, 0), 0, 0)

    prev_spec = lambda i: pl.BlockSpec((1, 1, 1, HOP), prev_map(i))
    return pl.pallas_call(
        _frame_kernel,
        out_shape=jax.ShapeDtypeStruct((B, num_frames, N_FFT), x.dtype),
        grid=grid,
        in_specs=[
            prev_spec(0),
            prev_spec(1),
            prev_spec(2),
            pl.BlockSpec((1, BF, HOP), lambda b, k: (b, k, 0)),
        ],
        out_specs=pl.BlockSpec((1, BF, N_FFT), lambda b, k: (b, k, 0)),
        compiler_params=pltpu.CompilerParams(
            dimension_semantics=("parallel", "arbitrary"),
        ),
        name="overlap_frame",
    )(x4, x4, x4, x3)


# speedup vs baseline: 6.2359x; 1.7060x over previous
import jax
import jax.numpy as jnp
from jax.experimental import pallas as pl
from jax.experimental.pallas import tpu as pltpu

N_FFT = 2048
HOP = 512
OVERLAP = N_FFT // HOP
PREV = OVERLAP - 1
BF = 128


def _frame_kernel(a0_ref, a1_ref, a2_ref, main_ref, o_ref):
    k = pl.program_id(1)
    rows = jnp.concatenate(
        [a0_ref[0, 0], a1_ref[0, 0], a2_ref[0, 0], main_ref[0]], axis=0
    )
    rid = jax.lax.broadcasted_iota(jnp.int32, rows.shape, 0)
    rows = jnp.where((k == 0) & (rid < PREV), 0.0, rows)
    o_ref[0] = jnp.concatenate(
        [rows[s:s + BF] for s in range(OVERLAP)], axis=1
    )


def kernel(x):
    B, T = x.shape
    num_chunks = T // HOP
    num_frames = num_chunks
    x3 = x.reshape(B, num_chunks, HOP)
    x4 = x.reshape(B, num_chunks, 1, HOP)
    grid = (B, num_frames // BF)

    def prev_map(i):
        return lambda b, k: (b, jnp.maximum(BF * k - PREV + i, 0), 0, 0)

    prev_spec = lambda i: pl.BlockSpec((1, 1, 1, HOP), prev_map(i))
    return pl.pallas_call(
        _frame_kernel,
        out_shape=jax.ShapeDtypeStruct((B, num_frames, N_FFT), x.dtype),
        grid=grid,
        in_specs=[
            prev_spec(0),
            prev_spec(1),
            prev_spec(2),
            pl.BlockSpec((1, BF, HOP), lambda b, k: (b, k, 0)),
        ],
        out_specs=pl.BlockSpec((1, BF, N_FFT), lambda b, k: (b, k, 0)),
        compiler_params=pltpu.CompilerParams(
            dimension_semantics=("parallel", "arbitrary"),
        ),
        name="overlap_frame",
    )(x4, x4, x4, x3)


# speedup vs baseline: 7.0909x; 1.1371x over previous
import jax
import jax.numpy as jnp
from jax.experimental import pallas as pl
from jax.experimental.pallas import tpu as pltpu

N_FFT = 2048
HOP = 512
OVERLAP = N_FFT // HOP
PREV = OVERLAP - 1
BF = 256


def _frame_kernel(a0_ref, a1_ref, a2_ref, main_ref, o_ref):
    k = pl.program_id(1)
    rows = jnp.concatenate(
        [a0_ref[0, 0], a1_ref[0, 0], a2_ref[0, 0], main_ref[0]], axis=0
    )
    rid = jax.lax.broadcasted_iota(jnp.int32, rows.shape, 0)
    rows = jnp.where((k == 0) & (rid < PREV), 0.0, rows)
    o_ref[0] = jnp.concatenate(
        [rows[s:s + BF] for s in range(OVERLAP)], axis=1
    )


def kernel(x):
    B, T = x.shape
    num_chunks = T // HOP
    num_frames = num_chunks
    x3 = x.reshape(B, num_chunks, HOP)
    x4 = x.reshape(B, num_chunks, 1, HOP)
    grid = (B, num_frames // BF)

    def prev_map(i):
        return lambda b, k: (b, jnp.maximum(BF * k - PREV + i, 0), 0, 0)

    prev_spec = lambda i: pl.BlockSpec((1, 1, 1, HOP), prev_map(i))
    return pl.pallas_call(
        _frame_kernel,
        out_shape=jax.ShapeDtypeStruct((B, num_frames, N_FFT), x.dtype),
        grid=grid,
        in_specs=[
            prev_spec(0),
            prev_spec(1),
            prev_spec(2),
            pl.BlockSpec((1, BF, HOP), lambda b, k: (b, k, 0)),
        ],
        out_specs=pl.BlockSpec((1, BF, N_FFT), lambda b, k: (b, k, 0)),
        compiler_params=pltpu.CompilerParams(
            dimension_semantics=("parallel", "arbitrary"),
        ),
        name="overlap_frame",
    )(x4, x4, x4, x3)


# speedup vs baseline: 7.5937x; 1.0709x over previous
import jax
import jax.numpy as jnp
from jax.experimental import pallas as pl
from jax.experimental.pallas import tpu as pltpu

N_FFT = 2048
HOP = 512
OVERLAP = N_FFT // HOP
PREV = OVERLAP - 1
BF = 512


def _frame_kernel(a0_ref, a1_ref, a2_ref, main_ref, o_ref):
    k = pl.program_id(1)
    rows = jnp.concatenate(
        [a0_ref[0, 0], a1_ref[0, 0], a2_ref[0, 0], main_ref[0]], axis=0
    )
    rid = jax.lax.broadcasted_iota(jnp.int32, rows.shape, 0)
    rows = jnp.where((k == 0) & (rid < PREV), 0.0, rows)
    o_ref[0] = jnp.concatenate(
        [rows[s:s + BF] for s in range(OVERLAP)], axis=1
    )


def kernel(x):
    B, T = x.shape
    num_chunks = T // HOP
    num_frames = num_chunks
    x3 = x.reshape(B, num_chunks, HOP)
    x4 = x.reshape(B, num_chunks, 1, HOP)
    grid = (B, num_frames // BF)

    def prev_map(i):
        return lambda b, k: (b, jnp.maximum(BF * k - PREV + i, 0), 0, 0)

    prev_spec = lambda i: pl.BlockSpec((1, 1, 1, HOP), prev_map(i))
    return pl.pallas_call(
        _frame_kernel,
        out_shape=jax.ShapeDtypeStruct((B, num_frames, N_FFT), x.dtype),
        grid=grid,
        in_specs=[
            prev_spec(0),
            prev_spec(1),
            prev_spec(2),
            pl.BlockSpec((1, BF, HOP), lambda b, k: (b, k, 0)),
        ],
        out_specs=pl.BlockSpec((1, BF, N_FFT), lambda b, k: (b, k, 0)),
        compiler_params=pltpu.CompilerParams(
            dimension_semantics=("parallel", "arbitrary"),
        ),
        name="overlap_frame",
    )(x4, x4, x4, x3)


# speedup vs baseline: 28.9712x; 3.8151x over previous
import jax
import jax.numpy as jnp
from jax.experimental import pallas as pl
from jax.experimental.pallas import tpu as pltpu

N_FFT = 2048
HOP = 512
OVERLAP = N_FFT // HOP
PREV = OVERLAP - 1
BF = 64
BB = 8


def _frame_kernel(p0_ref, p1_ref, p2_ref, main_ref, o_ref):
    k = pl.program_id(1)
    prev = jnp.stack([p0_ref[...], p1_ref[...], p2_ref[...]], axis=1)
    prev = jnp.where(k == 0, 0.0, prev)
    m3 = main_ref[...].reshape(BB, BF, HOP)
    rows = jnp.concatenate([prev, m3], axis=1)
    o_ref[...] = jnp.concatenate(
        [rows[:, s:s + BF, :] for s in range(OVERLAP)], axis=2
    )


def kernel(x):
    B, T = x.shape
    num_chunks = T // HOP
    num_frames = num_chunks
    grid = (B // BB, num_frames // BF)

    def prev_map(i):
        return lambda bb, k: (bb, jnp.maximum(BF * k - PREV + i, 0))

    prev_spec = lambda i: pl.BlockSpec((BB, HOP), prev_map(i))
    return pl.pallas_call(
        _frame_kernel,
        out_shape=jax.ShapeDtypeStruct((B, num_frames, N_FFT), x.dtype),
        grid=grid,
        in_specs=[
            prev_spec(0),
            prev_spec(1),
            prev_spec(2),
            pl.BlockSpec((BB, BF * HOP), lambda bb, k: (bb, k)),
        ],
        out_specs=pl.BlockSpec((BB, BF, N_FFT), lambda bb, k: (bb, k, 0)),
        compiler_params=pltpu.CompilerParams(
            dimension_semantics=("parallel", "arbitrary"),
        ),
        name="overlap_frame",
    )(x, x, x, x)


# speedup vs baseline: 31.0870x; 1.0730x over previous
import jax
import jax.numpy as jnp
from jax.experimental import pallas as pl
from jax.experimental.pallas import tpu as pltpu

N_FFT = 2048
HOP = 512
OVERLAP = N_FFT // HOP
PREV = OVERLAP - 1
BF = 128
BB = 8


def _frame_kernel(p0_ref, p1_ref, p2_ref, main_ref, o_ref):
    k = pl.program_id(1)
    prev = jnp.stack([p0_ref[...], p1_ref[...], p2_ref[...]], axis=1)
    prev = jnp.where(k == 0, 0.0, prev)
    m3 = main_ref[...].reshape(BB, BF, HOP)
    rows = jnp.concatenate([prev, m3], axis=1)
    o_ref[...] = jnp.concatenate(
        [rows[:, s:s + BF, :] for s in range(OVERLAP)], axis=2
    )


def kernel(x):
    B, T = x.shape
    num_chunks = T // HOP
    num_frames = num_chunks
    grid = (B // BB, num_frames // BF)

    def prev_map(i):
        return lambda bb, k: (bb, jnp.maximum(BF * k - PREV + i, 0))

    prev_spec = lambda i: pl.BlockSpec((BB, HOP), prev_map(i))
    return pl.pallas_call(
        _frame_kernel,
        out_shape=jax.ShapeDtypeStruct((B, num_frames, N_FFT), x.dtype),
        grid=grid,
        in_specs=[
            prev_spec(0),
            prev_spec(1),
            prev_spec(2),
            pl.BlockSpec((BB, BF * HOP), lambda bb, k: (bb, k)),
        ],
        out_specs=pl.BlockSpec((BB, BF, N_FFT), lambda bb, k: (bb, k, 0)),
        compiler_params=pltpu.CompilerParams(
            dimension_semantics=("parallel", "arbitrary"),
        ),
        name="overlap_frame",
    )(x, x, x, x)


# speedup vs baseline: 31.8453x; 1.0244x over previous
import jax
import jax.numpy as jnp
from jax.experimental import pallas as pl
from jax.experimental.pallas import tpu as pltpu

N_FFT = 2048
HOP = 512
OVERLAP = N_FFT // HOP
PREV = OVERLAP - 1
BF = 256
BB = 8


def _frame_kernel(p0_ref, p1_ref, p2_ref, main_ref, o_ref):
    k = pl.program_id(1)
    prev = jnp.stack([p0_ref[...], p1_ref[...], p2_ref[...]], axis=1)
    prev = jnp.where(k == 0, 0.0, prev)
    m3 = main_ref[...].reshape(BB, BF, HOP)
    rows = jnp.concatenate([prev, m3], axis=1)
    o_ref[...] = jnp.concatenate(
        [rows[:, s:s + BF, :] for s in range(OVERLAP)], axis=2
    )


def kernel(x):
    B, T = x.shape
    num_chunks = T // HOP
    num_frames = num_chunks
    grid = (B // BB, num_frames // BF)

    def prev_map(i):
        return lambda bb, k: (bb, jnp.maximum(BF * k - PREV + i, 0))

    prev_spec = lambda i: pl.BlockSpec((BB, HOP), prev_map(i))
    return pl.pallas_call(
        _frame_kernel,
        out_shape=jax.ShapeDtypeStruct((B, num_frames, N_FFT), x.dtype),
        grid=grid,
        in_specs=[
            prev_spec(0),
            prev_spec(1),
            prev_spec(2),
            pl.BlockSpec((BB, BF * HOP), lambda bb, k: (bb, k)),
        ],
        out_specs=pl.BlockSpec((BB, BF, N_FFT), lambda bb, k: (bb, k, 0)),
        compiler_params=pltpu.CompilerParams(
            dimension_semantics=("parallel", "arbitrary"),
            vmem_limit_bytes=56 * 1024 * 1024,
        ),
        name="overlap_frame",
    )(x, x, x, x)
